# Initial kernel scaffold; baseline (speedup 1.0000x reference)
import jax
import jax.numpy as jnp
from jax.experimental import pallas as pl


def kernel(feature_map, anchors):
    raise NotImplementedError("write your pallas kernel here")


# speedup vs baseline: 1.0164x; 1.0164x over previous
import functools

import jax
import jax.numpy as jnp
from jax import lax
from jax.experimental import pallas as pl
from jax.experimental.pallas import tpu as pltpu
from jax.experimental.pallas import tpu_sc as plsc

_COUNTS = 1000
_CB = 32
_N = 20000
_NPAD = 20480
_NW = 32
_PER_W = _NPAD // _NW
_MAGIC = 12582912.0


def _sat_body(x_ref, s_ref, acc_ref, t_ref):
    i = pl.program_id(0)

    @pl.when(i == 0)
    def _():
        acc_ref[...] = jnp.zeros_like(acc_ref)

    acc0 = acc_ref[...]

    def fold(c, acc):
        xc = x_ref[c]
        mc = jnp.mean(xc)
        return acc + jnp.abs(xc - mc)

    acc_ref[...] = jax.lax.fori_loop(0, _CB, fold, acc0)

    @pl.when(i == (256 // _CB) - 1)
    def _():
        def vfold(r, acc):
            acc = acc + acc_ref[pl.ds(r, 1), :]
            t_ref[pl.ds(r, 1), :] = acc
            return acc

        jax.lax.fori_loop(0, 128, vfold, jnp.zeros((1, 128), jnp.float32),
                          unroll=8)
        acc_ref[...] = jnp.transpose(t_ref[...])

        def hfold(r, acc):
            acc = acc + acc_ref[pl.ds(r, 1), :]
            t_ref[pl.ds(r, 1), :] = acc
            return acc

        jax.lax.fori_loop(0, 128, hfold, jnp.zeros((1, 128), jnp.float32),
                          unroll=8)
        s_ref[...] = jnp.zeros_like(s_ref)
        s_ref[1:129, 1:129] = jnp.transpose(t_ref[...])


def _sat(feature_map):
    fm = feature_map.reshape(256, 128, 128)
    S = pl.pallas_call(
        _sat_body,
        grid=(256 // _CB,),
        in_specs=[pl.BlockSpec((_CB, 128, 128), lambda i: (i, 0, 0))],
        out_specs=pl.BlockSpec((129, 129), lambda i: (0, 0)),
        out_shape=jax.ShapeDtypeStruct((129, 129), jnp.float32),
        scratch_shapes=[pltpu.VMEM((128, 128), jnp.float32),
                        pltpu.VMEM((128, 128), jnp.float32)],
    )(fm)
    return S[None]


def _heat_sc_body(s_hbm, an_hbm, out_hbm, s_v, a_v, heat_v):
    wid = lax.axis_index("s") * 2 + lax.axis_index("c")
    base = wid * _PER_W
    pltpu.sync_copy(s_hbm, s_v)
    for comp in range(4):
        pltpu.sync_copy(an_hbm.at[comp, pl.ds(base, _PER_W)], a_v.at[comp])

    def step(j, _):
        off = j * 16

        def coord(comp):
            v = a_v[comp, pl.ds(off, 16)] * 128.0
            r = (v + _MAGIC) - _MAGIC
            return jnp.clip(r.astype(jnp.int32), 0, 128)

        y1 = coord(0)
        x1 = coord(1)
        y2 = coord(2)
        x2 = coord(3)
        r1 = y1 * 129
        r2 = y2 * 129
        g22 = plsc.load_gather(s_v, [r2 + x2])
        g12 = plsc.load_gather(s_v, [r1 + x2])
        g21 = plsc.load_gather(s_v, [r2 + x1])
        g11 = plsc.load_gather(s_v, [r1 + x1])
        s = ((g22 - g12) - g21) + g11
        numel = jnp.maximum(y2 - y1, 0) * jnp.maximum(x2 - x1, 0)
        heat = s / numel.astype(jnp.float32)
        gidx = base + off + lax.iota(jnp.int32, 16)
        heat = jnp.where(gidx < _N, heat, jnp.float32(-3.0e38))
        heat_v[pl.ds(off, 16)] = heat
        return 0

    lax.fori_loop(0, _PER_W // 16, step, 0)
    pltpu.sync_copy(heat_v, out_hbm.at[pl.ds(base, _PER_W)])


def _heat_sc(S2d, an4):
    mesh = plsc.VectorSubcoreMesh(core_axis_name="c", subcore_axis_name="s")
    k = functools.partial(
        pl.kernel,
        mesh=mesh,
        out_type=jax.ShapeDtypeStruct((_NPAD,), jnp.float32),
        scratch_types=[
            pltpu.VMEM((129 * 129,), jnp.float32),
            pltpu.VMEM((4, _PER_W), jnp.float32),
            pltpu.VMEM((_PER_W,), jnp.float32),
        ],
        compiler_params=pltpu.CompilerParams(needs_layout_passes=False),
    )(_heat_sc_body)
    return k(S2d, an4)


def kernel(feature_map, anchors):
    counts = _COUNTS
    B, C, H, W = feature_map.shape
    stride = jnp.array([H, W, H, W], dtype=jnp.float32)
    a = anchors * stride

    S = _sat(feature_map)
    an4 = jnp.pad(anchors[0].T, ((0, 0), (0, _NPAD - _N)))
    heat = _heat_sc(S[0].reshape(129 * 129), an4)[: _N][None]

    _, idx = jax.lax.top_k(heat, counts)
    gidx = jnp.broadcast_to(idx[:, :, None], (B, counts, 4))
    proposals = jnp.take_along_axis(a, gidx, axis=1) / stride
    return proposals


# speedup vs baseline: 1.1002x; 1.0825x over previous
import functools

import jax
import jax.numpy as jnp
from jax import lax
from jax.experimental import pallas as pl
from jax.experimental.pallas import tpu as pltpu
from jax.experimental.pallas import tpu_sc as plsc

_COUNTS = 1000
_CB = 32
_N = 20000
_NPAD = 20480
_NW = 32
_PER_W = _NPAD // _NW
_MAGIC = 12582912.0


def _sat_body(x_ref, m_ref, s_ref, acc_ref, t_ref):
    i = pl.program_id(0)

    @pl.when(i == 0)
    def _():
        acc_ref[...] = jnp.zeros_like(acc_ref)

    acc0 = acc_ref[...]

    def fold(c, acc):
        xc = x_ref[c]
        mc = m_ref[pl.ds(c, 1), :]
        return acc + jnp.abs(xc - mc)

    acc_ref[...] = jax.lax.fori_loop(0, _CB, fold, acc0)

    @pl.when(i == (256 // _CB) - 1)
    def _():
        def vfold(r, acc):
            acc = acc + acc_ref[pl.ds(r, 1), :]
            t_ref[pl.ds(r, 1), :] = acc
            return acc

        jax.lax.fori_loop(0, 128, vfold, jnp.zeros((1, 128), jnp.float32),
                          unroll=8)
        acc_ref[...] = jnp.transpose(t_ref[...])

        def hfold(r, acc):
            acc = acc + acc_ref[pl.ds(r, 1), :]
            t_ref[pl.ds(r, 1), :] = acc
            return acc

        jax.lax.fori_loop(0, 128, hfold, jnp.zeros((1, 128), jnp.float32),
                          unroll=8)
        s_ref[...] = jnp.zeros_like(s_ref)
        s_ref[1:129, 1:129] = jnp.transpose(t_ref[...])


def _sat(feature_map, m):
    fm = feature_map.reshape(256, 128, 128)
    S = pl.pallas_call(
        _sat_body,
        grid=(256 // _CB,),
        in_specs=[pl.BlockSpec((_CB, 128, 128), lambda i: (i, 0, 0)),
                  pl.BlockSpec((_CB, 1), lambda i: (i, 0))],
        out_specs=pl.BlockSpec((129, 129), lambda i: (0, 0)),
        out_shape=jax.ShapeDtypeStruct((129, 129), jnp.float32),
        scratch_shapes=[pltpu.VMEM((128, 128), jnp.float32),
                        pltpu.VMEM((128, 128), jnp.float32)],
    )(fm, m.reshape(256, 1))
    return S[None]


def _heat_sc_body(s_hbm, an_hbm, out_hbm, s_v, a_v, heat_v):
    wid = lax.axis_index("s") * 2 + lax.axis_index("c")
    base = wid * _PER_W
    pltpu.sync_copy(s_hbm, s_v)
    for comp in range(4):
        pltpu.sync_copy(an_hbm.at[comp, pl.ds(base, _PER_W)], a_v.at[comp])

    def step(j, _):
        off = j * 16

        def coord(comp):
            v = a_v[comp, pl.ds(off, 16)] * 128.0
            r = (v + _MAGIC) - _MAGIC
            return jnp.clip(r.astype(jnp.int32), 0, 128)

        y1 = coord(0)
        x1 = coord(1)
        y2 = coord(2)
        x2 = coord(3)
        r1 = y1 * 129
        r2 = y2 * 129
        g22 = plsc.load_gather(s_v, [r2 + x2])
        g12 = plsc.load_gather(s_v, [r1 + x2])
        g21 = plsc.load_gather(s_v, [r2 + x1])
        g11 = plsc.load_gather(s_v, [r1 + x1])
        s = ((g22 - g12) - g21) + g11
        numel = jnp.maximum(y2 - y1, 0) * jnp.maximum(x2 - x1, 0)
        heat = s / numel.astype(jnp.float32)
        gidx = base + off + lax.iota(jnp.int32, 16)
        heat = jnp.where(gidx < _N, heat, jnp.float32(-3.0e38))
        heat_v[pl.ds(off, 16)] = heat
        return 0

    lax.fori_loop(0, _PER_W // 16, step, 0)
    pltpu.sync_copy(heat_v, out_hbm.at[pl.ds(base, _PER_W)])


def _heat_sc(S1d, an4):
    mesh = plsc.VectorSubcoreMesh(core_axis_name="c", subcore_axis_name="s")
    k = functools.partial(
        pl.kernel,
        mesh=mesh,
        out_type=jax.ShapeDtypeStruct((_NPAD,), jnp.float32),
        scratch_types=[
            pltpu.VMEM((129 * 129,), jnp.float32),
            pltpu.VMEM((4, _PER_W), jnp.float32),
            pltpu.VMEM((_PER_W,), jnp.float32),
        ],
        compiler_params=pltpu.CompilerParams(needs_layout_passes=False),
    )(_heat_sc_body)
    return k(S1d, an4)


def kernel(feature_map, anchors):
    counts = _COUNTS
    B, C, H, W = feature_map.shape
    stride = jnp.array([H, W, H, W], dtype=jnp.float32)
    a = anchors * stride

    m = jnp.mean(feature_map, axis=(2, 3), keepdims=True)
    S = _sat(feature_map, m.reshape(256))
    an4 = jnp.pad(anchors[0].T, ((0, 0), (0, _NPAD - _N)))
    heat = _heat_sc(S[0].reshape(129 * 129), an4)[: _N][None]

    _, idx = jax.lax.top_k(heat, counts)
    gidx = jnp.broadcast_to(idx[:, :, None], (B, counts, 4))
    proposals = jnp.take_along_axis(a, gidx, axis=1) / stride
    return proposals


# speedup vs baseline: 1.4910x; 1.3552x over previous
import functools

import jax
import jax.numpy as jnp
from jax import lax
from jax.experimental import pallas as pl
from jax.experimental.pallas import tpu as pltpu
from jax.experimental.pallas import tpu_sc as plsc

_COUNTS = 1000
_CB = 32
_N = 20000
_NPAD = 20480
_NW = 32
_PER_W = _NPAD // _NW
_MAGIC = 12582912.0


def _sat_body(x_ref, m_ref, s_ref, acc_ref, t_ref):
    i = pl.program_id(0)

    @pl.when(i == 0)
    def _():
        acc_ref[...] = jnp.zeros_like(acc_ref)

    acc0 = acc_ref[...]

    def fold(c, acc):
        xc = x_ref[c]
        mc = m_ref[pl.ds(c, 1), :]
        return acc + jnp.abs(xc - mc)

    acc_ref[...] = jax.lax.fori_loop(0, _CB, fold, acc0)

    @pl.when(i == (256 // _CB) - 1)
    def _():
        def vfold(r, acc):
            acc = acc + acc_ref[pl.ds(r, 1), :]
            t_ref[pl.ds(r, 1), :] = acc
            return acc

        jax.lax.fori_loop(0, 128, vfold, jnp.zeros((1, 128), jnp.float32),
                          unroll=8)
        acc_ref[...] = jnp.transpose(t_ref[...])

        def hfold(r, acc):
            acc = acc + acc_ref[pl.ds(r, 1), :]
            t_ref[pl.ds(r, 1), :] = acc
            return acc

        jax.lax.fori_loop(0, 128, hfold, jnp.zeros((1, 128), jnp.float32),
                          unroll=8)
        s_ref[...] = jnp.zeros_like(s_ref)
        s_ref[1:129, 1:129] = jnp.transpose(t_ref[...])


def _sat(feature_map, m):
    fm = feature_map.reshape(256, 128, 128)
    S = pl.pallas_call(
        _sat_body,
        grid=(256 // _CB,),
        in_specs=[pl.BlockSpec((_CB, 128, 128), lambda i: (i, 0, 0)),
                  pl.BlockSpec((_CB, 1), lambda i: (i, 0))],
        out_specs=pl.BlockSpec((129, 129), lambda i: (0, 0)),
        out_shape=jax.ShapeDtypeStruct((129, 129), jnp.float32),
        scratch_shapes=[pltpu.VMEM((128, 128), jnp.float32),
                        pltpu.VMEM((128, 128), jnp.float32)],
    )(fm, m.reshape(256, 1))
    return S[None]


def _heat_sc_body(s_hbm, an_hbm, out_hbm, s_v, a_v, heat_v):
    wid = lax.axis_index("s") * 2 + lax.axis_index("c")
    base = wid * _PER_W
    pltpu.sync_copy(s_hbm, s_v)
    for comp in range(4):
        pltpu.sync_copy(an_hbm.at[comp, pl.ds(base, _PER_W)], a_v.at[comp])

    def step(j, _):
        off = j * 16

        def coord(comp):
            v = a_v[comp, pl.ds(off, 16)] * 128.0
            r = (v + _MAGIC) - _MAGIC
            return jnp.clip(r.astype(jnp.int32), 0, 128)

        y1 = coord(0)
        x1 = coord(1)
        y2 = coord(2)
        x2 = coord(3)
        r1 = y1 * 129
        r2 = y2 * 129
        g22 = plsc.load_gather(s_v, [r2 + x2])
        g12 = plsc.load_gather(s_v, [r1 + x2])
        g21 = plsc.load_gather(s_v, [r2 + x1])
        g11 = plsc.load_gather(s_v, [r1 + x1])
        s = ((g22 - g12) - g21) + g11
        numel = jnp.maximum(y2 - y1, 0) * jnp.maximum(x2 - x1, 0)
        heat = s / numel.astype(jnp.float32)
        gidx = base + off + lax.iota(jnp.int32, 16)
        heat = jnp.where(gidx < _N, heat, jnp.float32(-3.0e38))
        heat_v[pl.ds(off, 16)] = heat
        return 0

    lax.fori_loop(0, _PER_W // 16, step, 0)
    pltpu.sync_copy(heat_v, out_hbm.at[pl.ds(base, _PER_W)])


def _heat_sc(S1d, an4):
    mesh = plsc.VectorSubcoreMesh(core_axis_name="c", subcore_axis_name="s")
    k = functools.partial(
        pl.kernel,
        mesh=mesh,
        out_type=jax.ShapeDtypeStruct((_NPAD,), jnp.float32),
        scratch_types=[
            pltpu.VMEM((129 * 129,), jnp.float32),
            pltpu.VMEM((4, _PER_W), jnp.float32),
            pltpu.VMEM((_PER_W,), jnp.float32),
        ],
        compiler_params=pltpu.CompilerParams(needs_layout_passes=False),
    )(_heat_sc_body)
    return k(S1d, an4)


_INT_MIN = -2147483648


def _dot(a, b, dn=(((1,), (0,)), ((), ()))):
    return jax.lax.dot_general(a, b, dimension_numbers=dn,
                               precision=jax.lax.Precision.HIGHEST,
                               preferred_element_type=jnp.float32)


def _sel_body(h_ref, dst_ref):
    h = h_ref[...]
    bits = jax.lax.bitcast_convert_type(h, jnp.int32)
    key = bits ^ (jax.lax.shift_right_arithmetic(bits, 31) & 0x7FFFFFFF)

    def bstep(k, tu):
        bit = jax.lax.shift_left(jnp.int32(1), 31 - k)
        cand = (tu | bit) ^ _INT_MIN
        cnt = jnp.sum((key >= cand).astype(jnp.int32))
        return jnp.where(cnt >= _COUNTS, tu | bit, tu)

    tu = jax.lax.fori_loop(0, 32, bstep, jnp.int32(0))
    t = tu ^ _INT_MIN
    gt = key > t
    eq = key == t
    r = (_COUNTS - jnp.sum(gt.astype(jnp.int32))).astype(jnp.float32)

    im = jax.lax.broadcasted_iota(jnp.int32, (128, 128), 0)
    il = jax.lax.broadcasted_iota(jnp.int32, (128, 128), 1)
    lower = (im < il).astype(jnp.float32)
    ir = jax.lax.broadcasted_iota(jnp.int32, (160, 160), 0)
    jr = jax.lax.broadcasted_iota(jnp.int32, (160, 160), 1)
    rtri = (jr < ir).astype(jnp.float32)

    def exclusive_prefix(mask_f):
        ex_in = _dot(mask_f, lower)
        rowtot = jnp.sum(mask_f, axis=1, keepdims=True)
        rowoff = _dot(rtri, rowtot)
        return rowoff + ex_in

    eqf = eq.astype(jnp.float32)
    eqrank = exclusive_prefix(eqf)
    sel = gt | (eq & (eqrank < r))
    self_f = sel.astype(jnp.float32)
    dst = exclusive_prefix(self_f)
    dst_ref[...] = jnp.where(sel, dst, 4000.0)


def _select(heat2d):
    return pl.pallas_call(
        _sel_body,
        out_shape=jax.ShapeDtypeStruct((160, 128), jnp.float32),
    )(heat2d)


def _order_body(d_ref, p_ref, out_ref, c_ref):
    i = pl.program_id(0)

    @pl.when(i == 0)
    def _():
        c_ref[...] = jnp.zeros_like(c_ref)

    d = d_ref[...]
    p = p_ref[...]
    kio = jax.lax.broadcasted_iota(jnp.int32, (1, 128), 1)
    for kc in range(8):
        oh = (d == (kio + kc * 128).astype(jnp.float32)).astype(jnp.float32)
        part = _dot(oh, p, dn=(((0,), (0,)), ((), ())))
        c_ref[kc * 128:(kc + 1) * 128, :] += part

    @pl.when(i == (_NPAD // 1024) - 1)
    def _():
        cm = c_ref[...]
        rio = jax.lax.broadcasted_iota(jnp.int32, (1024, 8), 0)
        cio = jax.lax.broadcasted_iota(jnp.int32, (1024, 8), 1)
        pad = rio >= _COUNTS
        cm = jnp.where(pad & (cio == 4), -3.2e38, cm)
        cm = jnp.where(pad & (cio == 5), 3.0e7, cm)
        cmt = jnp.transpose(cm)
        ckey = cm[:, 4:5]
        cidx = cm[:, 5:6]
        kcr = cmt[4:5, :]
        icr = cmt[5:6, :]
        prec = (kcr > ckey) | ((kcr == ckey) & (icr < cidx))
        posf = jnp.sum(prec.astype(jnp.float32), axis=1, keepdims=True)
        pio = jax.lax.broadcasted_iota(jnp.int32, (1, 128), 1)
        for pc in range(8):
            oh2 = (posf == (pio + pc * 128).astype(jnp.float32)
                   ).astype(jnp.float32)
            blk = _dot(oh2, cm, dn=(((0,), (0,)), ((), ())))
            out_ref[pc * 128:(pc + 1) * 128, :] = blk


def _order(dcol, payload):
    nblk = _NPAD // 1024
    return pl.pallas_call(
        _order_body,
        grid=(nblk,),
        in_specs=[pl.BlockSpec((1024, 1), lambda i: (i, 0)),
                  pl.BlockSpec((1024, 8), lambda i: (i, 0))],
        out_specs=pl.BlockSpec((1024, 8), lambda i: (0, 0)),
        out_shape=jax.ShapeDtypeStruct((1024, 8), jnp.float32),
        scratch_shapes=[pltpu.VMEM((1024, 8), jnp.float32)],
    )(dcol, payload)


def kernel(feature_map, anchors):
    counts = _COUNTS
    B, C, H, W = feature_map.shape

    m = jnp.mean(feature_map, axis=(2, 3), keepdims=True)
    S = _sat(feature_map, m.reshape(256))
    an0 = anchors[0]
    an4 = jnp.pad(an0.T, ((0, 0), (0, _NPAD - _N)))
    heat_pad = _heat_sc(S[0].reshape(129 * 129), an4)

    dcol = _select(heat_pad.reshape(160, 128)).reshape(_NPAD, 1)
    payload = jnp.concatenate(
        [jnp.pad(an0, ((0, _NPAD - _N), (0, 0))),
         heat_pad[:, None],
         jnp.arange(_NPAD, dtype=jnp.float32)[:, None],
         jnp.zeros((_NPAD, 2), jnp.float32)], axis=1)
    out = _order(dcol, payload)
    return out[:counts, :4][None]
